# baseline (speedup 1.0000x reference)
import functools

import jax
import jax.numpy as jnp
from jax import lax
from jax.experimental import pallas as pl
from jax.experimental.pallas import tpu as pltpu
from jax.experimental.pallas import tpu_sc as plsc

DIM = 64
P = 20
NEG = 50
REAL = P + NEG
IDXW = 72
DOTW = 80
NC = 2
NS = 16
NW = NC * NS
L = 16
NBUF = 8


@functools.lru_cache(maxsize=None)
def _sc_dots_fn(B: int):
    CH = B // NW

    mesh = plsc.VectorSubcoreMesh(core_axis_name="c", subcore_axis_name="s")

    @functools.partial(
        pl.kernel,
        out_type=jax.ShapeDtypeStruct((B, DOTW), jnp.float32),
        mesh=mesh,
        scratch_types=[
            pltpu.VMEM((CH,), jnp.int32),
            pltpu.VMEM((CH, IDXW), jnp.int32),
            pltpu.VMEM((CH, DIM), jnp.float32),
            pltpu.VMEM((NBUF, IDXW, DIM), jnp.float32),
            pltpu.VMEM((NBUF, DOTW), jnp.float32),
            pltpu.VMEM((L, L), jnp.float32),
            *([pltpu.SemaphoreType.DMA] * (2 * NBUF)),
        ],
        compiler_params=pltpu.CompilerParams(
            needs_layout_passes=False, use_tc_tiling_on_sc=False),
    )
    def sc_dots(center_hbm, ctx_hbm, in_emb, out_emb, dots_hbm,
                cidx_v, ctx_v, cmat_v, rows_v, dots_v, t_v, *sems):
        gsem = sems[:NBUF]
        osem = sems[NBUF:]
        wid = lax.axis_index("s") * NC + lax.axis_index("c")
        base = wid * CH
        lane = lax.iota(jnp.int32, L)

        pltpu.sync_copy(center_hbm.at[pl.ds(base, CH)], cidx_v)
        pltpu.sync_copy(ctx_hbm.at[pl.ds(base, CH)], ctx_v)
        for k in range(CH // 128):
            pltpu.sync_copy(in_emb.at[cidx_v.at[pl.ds(k * 128, 128)]],
                            cmat_v.at[pl.ds(k * 128, 128)])

        def start(j, b):
            pltpu.async_copy(out_emb.at[ctx_v.at[b]], rows_v.at[j], gsem[j])

        def wait(j, b):
            pltpu.make_async_copy(out_emb.at[ctx_v.at[b]], rows_v.at[j],
                                  gsem[j]).wait()

        def start_out(j, b):
            pltpu.async_copy(dots_v.at[j], dots_hbm.at[base + b], osem[j])

        def wait_out(j, b):
            pltpu.make_async_copy(dots_v.at[j], dots_hbm.at[base + b],
                                  osem[j]).wait()

        last = jnp.full((L,), L - 1, jnp.int32)

        def compute(j, b):
            c = [cmat_v[b, pl.ds(L * t, L)] for t in range(DIM // L)]
            for g in range(DOTW // L):
                for q in range(min(REAL - g * L, L)):
                    r = g * L + q
                    p = (rows_v[j, r, pl.ds(0, L)] * c[0]
                         + rows_v[j, r, pl.ds(L, L)] * c[1]
                         + rows_v[j, r, pl.ds(2 * L, L)] * c[2]
                         + rows_v[j, r, pl.ds(3 * L, L)] * c[3])
                    t_v[q, :] = plsc.cumsum(p)
                dots_v[j, pl.ds(g * L, L)] = plsc.load_gather(t_v, [lane, last])

        for j in range(NBUF):
            start(j, j)

        def body(i, carry):
            b0 = i * NBUF
            for j in range(NBUF):
                b = b0 + j
                wait(j, b)

                @pl.when(b >= NBUF)
                def _():
                    wait_out(j, b - NBUF)

                compute(j, b)
                start_out(j, b)

                @pl.when(b + NBUF < CH)
                def _():
                    start(j, b + NBUF)
            return carry

        lax.fori_loop(0, CH // NBUF, body, 0)
        for j in range(NBUF):
            wait_out(j, CH - NBUF + j)

    return sc_dots


def _tc_loss_body(d_ref, o_ref):
    x = d_ref[...]
    col = lax.broadcasted_iota(jnp.int32, x.shape, 1)
    y = x * jnp.where(col < P, 1.0, -1.0).astype(jnp.float32)
    ls = jnp.minimum(y, 0.0) - jnp.log1p(jnp.exp(-jnp.abs(y)))
    contrib = jnp.where(col < REAL, ls, 0.0)
    o_ref[...] = -jnp.sum(contrib, axis=1, keepdims=True)


@functools.lru_cache(maxsize=None)
def _tc_loss_fn(B: int):
    BT = 2048
    return pl.pallas_call(
        _tc_loss_body,
        grid=(B // BT,),
        in_specs=[pl.BlockSpec((BT, DOTW), lambda i: (i, 0))],
        out_specs=pl.BlockSpec((BT, 1), lambda i: (i, 0)),
        out_shape=jax.ShapeDtypeStruct((B, 1), jnp.float32),
    )


def kernel(center, positive, negative, in_emb, out_emb):
    B = center.shape[0]
    pad = jnp.zeros((B, IDXW - REAL), jnp.int32)
    ctx = jnp.concatenate(
        [positive.astype(jnp.int32), negative.astype(jnp.int32), pad], axis=1)
    dots = _sc_dots_fn(B)(center.astype(jnp.int32), ctx, in_emb, out_emb)
    return _tc_loss_fn(B)(dots)


# speedup vs baseline: 1.3604x; 1.3604x over previous
import functools

import jax
import jax.numpy as jnp
from jax import lax
from jax.experimental import pallas as pl
from jax.experimental.pallas import tpu as pltpu
from jax.experimental.pallas import tpu_sc as plsc

DIM = 64
P = 20
NEG = 50
REAL = P + NEG
IDXW = 72
DOTW = 80
NC = 2
NS = 16
NW = NC * NS
L = 16
NBUF = 8


@functools.lru_cache(maxsize=None)
def _sc_dots_fn(B: int):
    CH = B // NW

    mesh = plsc.VectorSubcoreMesh(core_axis_name="c", subcore_axis_name="s")

    @functools.partial(
        pl.kernel,
        out_type=jax.ShapeDtypeStruct((B, DOTW), jnp.float32),
        mesh=mesh,
        scratch_types=[
            pltpu.VMEM((CH,), jnp.int32),
            pltpu.VMEM((CH, IDXW), jnp.int32),
            pltpu.VMEM((CH, DIM), jnp.float32),
            pltpu.VMEM((NBUF, IDXW, DIM), jnp.float32),
            pltpu.VMEM((NBUF, DOTW), jnp.float32),
            pltpu.VMEM((L, L), jnp.float32),
            *([pltpu.SemaphoreType.DMA] * (2 * NBUF)),
        ],
        compiler_params=pltpu.CompilerParams(
            needs_layout_passes=False, use_tc_tiling_on_sc=False),
    )
    def sc_dots(center_hbm, ctx_hbm, in_emb, out_emb, dots_hbm,
                cidx_v, ctx_v, cmat_v, rows_v, dots_v, t_v, *sems):
        gsem = sems[:NBUF]
        osem = sems[NBUF:]
        wid = lax.axis_index("s") * NC + lax.axis_index("c")
        base = wid * CH
        lane = lax.iota(jnp.int32, L)

        pltpu.sync_copy(center_hbm.at[pl.ds(base, CH)], cidx_v)
        pltpu.sync_copy(ctx_hbm.at[pl.ds(base, CH)], ctx_v)
        for k in range(CH // 128):
            pltpu.sync_copy(in_emb.at[cidx_v.at[pl.ds(k * 128, 128)]],
                            cmat_v.at[pl.ds(k * 128, 128)])

        def start(j, b):
            pltpu.async_copy(out_emb.at[ctx_v.at[b]], rows_v.at[j], gsem[j])

        def wait(j, b):
            pltpu.make_async_copy(out_emb.at[ctx_v.at[b]], rows_v.at[j],
                                  gsem[j]).wait()

        def start_out(j, b):
            pltpu.async_copy(dots_v.at[j], dots_hbm.at[base + b], osem[j])

        def wait_out(j, b):
            pltpu.make_async_copy(dots_v.at[j], dots_hbm.at[base + b],
                                  osem[j]).wait()

        last = jnp.full((L,), L - 1, jnp.int32)

        def compute(j, b):
            c = [cmat_v[b, pl.ds(L * t, L)] for t in range(DIM // L)]
            for g in range(DOTW // L):
                for q in range(min(REAL - g * L, L)):
                    r = g * L + q
                    p = (rows_v[j, r, pl.ds(0, L)] * c[0]
                         + rows_v[j, r, pl.ds(L, L)] * c[1]
                         + rows_v[j, r, pl.ds(2 * L, L)] * c[2]
                         + rows_v[j, r, pl.ds(3 * L, L)] * c[3])
                    t_v[q, :] = plsc.cumsum(p)
                dots_v[j, pl.ds(g * L, L)] = plsc.load_gather(t_v, [lane, last])

        for j in range(NBUF):
            start(j, j)

        def body(i, carry):
            b0 = i * NBUF
            for j in range(NBUF):
                b = b0 + j
                wait(j, b)

                @pl.when(b >= NBUF)
                def _():
                    wait_out(j, b - NBUF)

                compute(j, b)
                start_out(j, b)

                @pl.when(b + NBUF < CH)
                def _():
                    start(j, b + NBUF)
            return carry

        lax.fori_loop(0, CH // NBUF, body, 0)
        for j in range(NBUF):
            wait_out(j, CH - NBUF + j)

    return sc_dots


def _tc_loss_body(d_ref, o_ref):
    x = d_ref[...]
    col = lax.broadcasted_iota(jnp.int32, x.shape, 1)
    y = x * jnp.where(col < P, 1.0, -1.0).astype(jnp.float32)
    ls = jnp.minimum(y, 0.0) - jnp.log1p(jnp.exp(-jnp.abs(y)))
    contrib = jnp.where(col < REAL, ls, 0.0)
    o_ref[...] = -jnp.sum(contrib, axis=1, keepdims=True)


@functools.lru_cache(maxsize=None)
def _tc_loss_fn(B: int):
    BT = 2048
    return pl.pallas_call(
        _tc_loss_body,
        grid=(B // BT,),
        in_specs=[pl.BlockSpec((BT, DOTW), lambda i: (i, 0))],
        out_specs=pl.BlockSpec((BT, 1), lambda i: (i, 0)),
        out_shape=jax.ShapeDtypeStruct((B, 1), jnp.float32),
    )


def kernel(center, positive, negative, in_emb, out_emb):
    B = center.shape[0]
    vocab = out_emb.shape[0]
    pad = (jnp.arange(B, dtype=jnp.int32)[:, None] * (IDXW - REAL)
           + jnp.arange(IDXW - REAL, dtype=jnp.int32)[None, :]) % vocab
    ctx = jnp.concatenate(
        [positive.astype(jnp.int32), negative.astype(jnp.int32), pad], axis=1)
    dots = _sc_dots_fn(B)(center.astype(jnp.int32), ctx, in_emb, out_emb)
    return _tc_loss_fn(B)(dots)
